# baseline (speedup 1.0000x reference)
import functools

import jax
import jax.numpy as jnp
from jax import lax
from jax.experimental import pallas as pl
from jax.experimental.pallas import tpu as pltpu
from jax.experimental.pallas import tpu_sc as plsc

CHUNK = 128
NBUF = 4
DP = 128


@functools.cache
def _build_detile(V, D):
    info = plsc.get_sparse_core_info()
    nw = info.num_cores * info.num_subcores
    nblk = (V + 127) // 128
    n_per_w = (nblk + nw - 1) // nw
    mesh = plsc.VectorSubcoreMesh(core_axis_name="c", subcore_axis_name="s")

    @functools.partial(
        pl.kernel,
        out_type=jax.ShapeDtypeStruct((V // 2, 128), jnp.float32),
        mesh=mesh,
        scratch_types=[
            pltpu.VMEM((D, 128), jnp.float32),
            pltpu.VMEM((64, 128), jnp.float32),
        ],
        compiler_params=pltpu.CompilerParams(use_tc_tiling_on_sc=True, needs_layout_passes=False),
    )
    def ka(wt_hbm, tail_hbm, out_hbm, stag, obuf):
        wid = lax.axis_index("s") * info.num_cores + lax.axis_index("c")
        iota = lax.iota(jnp.int32, 16)
        nfull = V // 128
        tail = V - nfull * 128

        def transpose_block(ngroups):
            for gi in range(ngroups):
                i0 = gi * 16
                rvec = (i0 + iota) >> 1
                cbase = ((i0 + iota) & 1) * 64

                def body(j, carry):
                    val = stag[j, pl.ds(i0, 16)]
                    plsc.store_scatter(obuf, [rvec, cbase + j], val)
                    return carry

                lax.fori_loop(0, D, body, 0)

        @pl.loop(0, n_per_w)
        def _(t):
            u = wid * n_per_w + t

            @pl.when(u < nfull)
            def _():
                c0 = u * 128
                pltpu.sync_copy(wt_hbm.at[:, pl.ds(c0, 128)], stag)
                transpose_block(8)
                pltpu.sync_copy(obuf, out_hbm.at[pl.ds(u * 64, 64)])

        if tail:
            @pl.when(wid == 0)
            def _():
                pltpu.sync_copy(tail_hbm, stag)
                for i in range(tail):
                    for g in range(D // 16):
                        obuf[i >> 1, pl.ds((i & 1) * 64 + g * 16, 16)] = (
                            stag[i, pl.ds(g * 16, 16)]
                        )
                pltpu.sync_copy(
                    obuf.at[pl.ds(0, tail // 2)],
                    out_hbm.at[pl.ds(nfull * 64, tail // 2)],
                )

    return ka


@functools.cache
def _build_gather(B, V, D):
    info = plsc.get_sparse_core_info()
    nw = info.num_cores * info.num_subcores
    assert B % (nw * CHUNK * NBUF) == 0
    b_per_w = B // nw
    n_groups = b_per_w // (CHUNK * NBUF)
    mesh = plsc.VectorSubcoreMesh(core_axis_name="c", subcore_axis_name="s")

    @functools.partial(
        pl.kernel,
        out_type=jax.ShapeDtypeStruct((B, DP), jnp.float32),
        mesh=mesh,
        scratch_types=[
            pltpu.VMEM((b_per_w,), jnp.int32),
            pltpu.VMEM((NBUF, CHUNK, D), jnp.float32),
            pltpu.VMEM((NBUF, CHUNK, DP), jnp.float32),
            pltpu.SemaphoreType.DMA((NBUF,)),
            pltpu.SemaphoreType.DMA((NBUF,)),
        ],
        compiler_params=pltpu.CompilerParams(use_tc_tiling_on_sc=False),
    )
    def kb(x_hbm, tab_hbm, out_hbm, idx_v, rows_v, wbuf, gsem, wsem):
        wid = lax.axis_index("s") * info.num_cores + lax.axis_index("c")
        base = wid * b_per_w
        pltpu.sync_copy(x_hbm.at[pl.ds(base, b_per_w)], idx_v)

        def gather(j, b):
            return pltpu.make_async_copy(
                tab_hbm.at[idx_v.at[pl.ds(j * CHUNK, CHUNK)]],
                rows_v.at[b],
                gsem.at[b],
            )

        def write(j, b):
            return pltpu.make_async_copy(
                wbuf.at[b],
                out_hbm.at[pl.ds(base + j * CHUNK, CHUNK)],
                wsem.at[b],
            )

        def repack(b):
            def body(i, carry):
                for g in range(D // 16):
                    wbuf[b, i, pl.ds(g * 16, 16)] = rows_v[b, i, pl.ds(g * 16, 16)]
                return carry

            lax.fori_loop(0, CHUNK, body, 0)

        for b in range(NBUF):
            gather(b, b).start()

        @pl.loop(0, n_groups)
        def _(g):
            j0 = g * NBUF
            for b in range(NBUF):
                gather(j0 + b, b).wait()
                repack(b)
                write(j0 + b, b).start()
            for b in range(NBUF):
                write(j0 + b, b).wait()

                @pl.when(g + 1 < n_groups)
                def _():
                    gather(j0 + NBUF + b, b).start()

    return kb


def kernel(x, W_embed):
    batch, hist = x.shape
    V, D = W_embed.shape
    flat = x.reshape(batch * hist).astype(jnp.int32)
    nfull = V // 128
    tail_pad = jnp.pad(W_embed[nfull * 128:], ((0, 0), (0, 128 - D)))
    pairs = _build_detile(V, D)(W_embed.T, tail_pad)
    tab = pairs.reshape(V, D)
    out_pad = _build_gather(batch * hist, V, D)(flat, tab)
    return out_pad.reshape(batch, hist, DP)[:, :, :D]


# speedup vs baseline: 1.2062x; 1.2062x over previous
import functools

import jax
import jax.numpy as jnp
from jax import lax
from jax.experimental import pallas as pl
from jax.experimental.pallas import tpu as pltpu
from jax.experimental.pallas import tpu_sc as plsc

CHUNK = 128
NBUF = 4
DP = 128


@functools.cache
def _build_detile(V, D):
    info = plsc.get_sparse_core_info()
    nw = info.num_cores * info.num_subcores
    nblk = (V + 127) // 128
    n_per_w = (nblk + nw - 1) // nw
    mesh = plsc.VectorSubcoreMesh(core_axis_name="c", subcore_axis_name="s")

    @functools.partial(
        pl.kernel,
        out_type=jax.ShapeDtypeStruct((V // 2, 128), jnp.float32),
        mesh=mesh,
        scratch_types=[
            pltpu.VMEM((2, D, 128), jnp.float32),
            pltpu.VMEM((2, 64, 128), jnp.float32),
            pltpu.SemaphoreType.DMA((2,)),
            pltpu.SemaphoreType.DMA((2,)),
        ],
        compiler_params=pltpu.CompilerParams(use_tc_tiling_on_sc=True, needs_layout_passes=False),
    )
    def ka(wt_hbm, tail_hbm, out_hbm, stag, obuf, gsem, wsem):
        wid = lax.axis_index("s") * info.num_cores + lax.axis_index("c")
        iota = lax.iota(jnp.int32, 16)
        nfull = V // 128
        tail = V - nfull * 128
        npw = (nfull + nw - 1) // nw

        def uclamp(t):
            return jnp.minimum(wid * npw + t, nfull - 1)

        def copy_in(t, p):
            c0 = uclamp(t) * 128
            return pltpu.make_async_copy(
                wt_hbm.at[:, pl.ds(c0, 128)], stag.at[p], gsem.at[p]
            )

        def copy_out(t, p):
            return pltpu.make_async_copy(
                obuf.at[p], out_hbm.at[pl.ds(uclamp(t) * 64, 64)], wsem.at[p]
            )

        rvecs = [((gi * 16 + iota) >> 1) for gi in range(8)]
        cbases = [((gi * 16 + iota) & 1) * 64 for gi in range(8)]

        def transpose_block(p):
            def body(j, carry):
                for gi in range(8):
                    val = stag[p, j, pl.ds(gi * 16, 16)]
                    plsc.store_scatter(obuf.at[p], [rvecs[gi], cbases[gi] + j], val)
                return carry

            lax.fori_loop(0, D, body, 0, unroll=4)

        def step(t, p, first):
            copy_in(t, p).wait()
            if not first:
                copy_out(t - 2, p).wait()
            transpose_block(p)
            copy_out(t, p).start()

        copy_in(0, 0).start()
        copy_in(1, 1).start()

        @pl.loop(0, (npw - 1) // 2)
        def _(h):
            for p in range(2):
                t = h * 2 + p

                @pl.when(h >= 1)
                def _():
                    copy_out(t - 2, p).wait()
                copy_in(t, p).wait()
                transpose_block(p)
                copy_out(t, p).start()

                @pl.when(t + 2 < npw)
                def _():
                    copy_in(t + 2, p).start()

        tlast = npw - 1
        copy_in(tlast, 0).wait()
        copy_out(tlast - 2, 0).wait()
        transpose_block(0)
        copy_out(tlast, 0).start()
        copy_out(tlast - 1, 1).wait()
        copy_out(tlast, 0).wait()

        if tail:
            @pl.when(wid == 0)
            def _():
                pltpu.sync_copy(tail_hbm, stag.at[0])
                for i in range(tail):
                    for g in range(D // 16):
                        obuf[0, i >> 1, pl.ds((i & 1) * 64 + g * 16, 16)] = (
                            stag[0, i, pl.ds(g * 16, 16)]
                        )
                pltpu.sync_copy(
                    obuf.at[0, pl.ds(0, tail // 2)],
                    out_hbm.at[pl.ds(nfull * 64, tail // 2)],
                )

    return ka


@functools.cache
def _build_gather(B, V, D):
    info = plsc.get_sparse_core_info()
    nw = info.num_cores * info.num_subcores
    assert B % (nw * CHUNK * NBUF) == 0
    b_per_w = B // nw
    n_groups = b_per_w // (CHUNK * NBUF)
    mesh = plsc.VectorSubcoreMesh(core_axis_name="c", subcore_axis_name="s")

    @functools.partial(
        pl.kernel,
        out_type=jax.ShapeDtypeStruct((B, DP), jnp.float32),
        mesh=mesh,
        scratch_types=[
            pltpu.VMEM((b_per_w,), jnp.int32),
            pltpu.VMEM((NBUF, CHUNK, D), jnp.float32),
            pltpu.VMEM((NBUF, CHUNK, DP), jnp.float32),
            pltpu.SemaphoreType.DMA((NBUF,)),
            pltpu.SemaphoreType.DMA((NBUF,)),
        ],
        compiler_params=pltpu.CompilerParams(use_tc_tiling_on_sc=False),
    )
    def kb(x_hbm, tab_hbm, out_hbm, idx_v, rows_v, wbuf, gsem, wsem):
        wid = lax.axis_index("s") * info.num_cores + lax.axis_index("c")
        base = wid * b_per_w
        pltpu.sync_copy(x_hbm.at[pl.ds(base, b_per_w)], idx_v)

        def gather(j, b):
            return pltpu.make_async_copy(
                tab_hbm.at[idx_v.at[pl.ds(j * CHUNK, CHUNK)]],
                rows_v.at[b],
                gsem.at[b],
            )

        def write(j, b):
            return pltpu.make_async_copy(
                wbuf.at[b],
                out_hbm.at[pl.ds(base + j * CHUNK, CHUNK)],
                wsem.at[b],
            )

        def repack(b):
            def body(i, carry):
                for k in range(8):
                    for g in range(D // 16):
                        wbuf[b, i * 8 + k, pl.ds(g * 16, 16)] = (
                            rows_v[b, i * 8 + k, pl.ds(g * 16, 16)]
                        )
                return carry

            lax.fori_loop(0, CHUNK // 8, body, 0)

        for b in range(NBUF):
            gather(b, b).start()

        @pl.loop(0, n_groups)
        def _(g):
            j0 = g * NBUF
            for b in range(NBUF):
                gather(j0 + b, b).wait()
                repack(b)
                write(j0 + b, b).start()
            for b in range(NBUF):
                write(j0 + b, b).wait()

                @pl.when(g + 1 < n_groups)
                def _():
                    gather(j0 + NBUF + b, b).start()

    return kb


def kernel(x, W_embed):
    batch, hist = x.shape
    V, D = W_embed.shape
    flat = x.reshape(batch * hist).astype(jnp.int32)
    nfull = V // 128
    tail_pad = jnp.pad(W_embed[nfull * 128:], ((0, 0), (0, 128 - D)))
    pairs = _build_detile(V, D)(W_embed.T, tail_pad)
    tab = pairs.reshape(V, D)
    out_pad = _build_gather(batch * hist, V, D)(flat, tab)
    return out_pad.reshape(batch, hist, DP)[:, :, :D]


# speedup vs baseline: 1.8560x; 1.5387x over previous
import functools

import jax
import jax.numpy as jnp
from jax import lax
from jax.experimental import pallas as pl
from jax.experimental.pallas import tpu as pltpu
from jax.experimental.pallas import tpu_sc as plsc

CHUNK = 128
NBUF = 4
DP = 128


@functools.cache
def _build_detile(V, D):
    info = plsc.get_sparse_core_info()
    nw = info.num_cores * info.num_subcores
    nblk = (V + 127) // 128
    n_per_w = (nblk + nw - 1) // nw
    mesh = plsc.VectorSubcoreMesh(core_axis_name="c", subcore_axis_name="s")

    @functools.partial(
        pl.kernel,
        out_type=jax.ShapeDtypeStruct((V // 2, 128), jnp.float32),
        mesh=mesh,
        scratch_types=[
            pltpu.VMEM((2, D, 128), jnp.float32),
            pltpu.VMEM((2, 64, 128), jnp.float32),
            pltpu.SemaphoreType.DMA((2,)),
            pltpu.SemaphoreType.DMA((2,)),
        ],
        compiler_params=pltpu.CompilerParams(use_tc_tiling_on_sc=True, needs_layout_passes=False),
    )
    def ka(wt_hbm, tail_hbm, out_hbm, stag, obuf, gsem, wsem):
        wid = lax.axis_index("s") * info.num_cores + lax.axis_index("c")
        iota = lax.iota(jnp.int32, 16)
        nfull = V // 128
        tail = V - nfull * 128
        npw = (nfull + nw - 1) // nw

        def uclamp(t):
            return jnp.minimum(wid * npw + t, nfull - 1)

        def copy_in(t, p):
            c0 = uclamp(t) * 128
            return pltpu.make_async_copy(
                wt_hbm.at[:, pl.ds(c0, 128)], stag.at[p], gsem.at[p]
            )

        def copy_out(t, p):
            return pltpu.make_async_copy(
                obuf.at[p], out_hbm.at[pl.ds(uclamp(t) * 64, 64)], wsem.at[p]
            )

        rots = [(iota + s) % 16 for s in range(16)]

        def transpose_block(p):
            def body(ib4, carry):
                ib = ib4 * 16
                r0 = ib4 * 8
                for jb in range(0, D, 16):
                    jvec = jb + iota
                    for s in range(16):
                        ivec = ib + rots[s]
                        val = plsc.load_gather(stag.at[p], [jvec, ivec])
                        rvec = r0 + (rots[s] >> 1)
                        cvec = (rots[s] & 1) * 64 + jvec
                        plsc.store_scatter(obuf.at[p], [rvec, cvec], val)
                return carry

            lax.fori_loop(0, 8, body, 0)

        def step(t, p, first):
            copy_in(t, p).wait()
            if not first:
                copy_out(t - 2, p).wait()
            transpose_block(p)
            copy_out(t, p).start()

        copy_in(0, 0).start()
        copy_in(1, 1).start()

        @pl.loop(0, (npw - 1) // 2)
        def _(h):
            for p in range(2):
                t = h * 2 + p

                @pl.when(h >= 1)
                def _():
                    copy_out(t - 2, p).wait()
                copy_in(t, p).wait()
                transpose_block(p)
                copy_out(t, p).start()

                @pl.when(t + 2 < npw)
                def _():
                    copy_in(t + 2, p).start()

        tlast = npw - 1
        copy_in(tlast, 0).wait()
        copy_out(tlast - 2, 0).wait()
        transpose_block(0)
        copy_out(tlast, 0).start()
        copy_out(tlast - 1, 1).wait()
        copy_out(tlast, 0).wait()

        if tail:
            @pl.when(wid == 0)
            def _():
                pltpu.sync_copy(tail_hbm, stag.at[0])
                for i in range(tail):
                    for g in range(D // 16):
                        obuf[0, i >> 1, pl.ds((i & 1) * 64 + g * 16, 16)] = (
                            stag[0, i, pl.ds(g * 16, 16)]
                        )
                pltpu.sync_copy(
                    obuf.at[0, pl.ds(0, tail // 2)],
                    out_hbm.at[pl.ds(nfull * 64, tail // 2)],
                )

    return ka


@functools.cache
def _build_gather(B, V, D):
    info = plsc.get_sparse_core_info()
    nw = info.num_cores * info.num_subcores
    assert B % (nw * CHUNK * NBUF) == 0
    b_per_w = B // nw
    n_groups = b_per_w // (CHUNK * NBUF)
    mesh = plsc.VectorSubcoreMesh(core_axis_name="c", subcore_axis_name="s")

    @functools.partial(
        pl.kernel,
        out_type=jax.ShapeDtypeStruct((B, DP), jnp.float32),
        mesh=mesh,
        scratch_types=[
            pltpu.VMEM((b_per_w,), jnp.int32),
            pltpu.VMEM((NBUF, CHUNK, D), jnp.float32),
            pltpu.VMEM((NBUF, CHUNK, DP), jnp.float32),
            pltpu.SemaphoreType.DMA((NBUF,)),
            pltpu.SemaphoreType.DMA((NBUF,)),
        ],
        compiler_params=pltpu.CompilerParams(use_tc_tiling_on_sc=False),
    )
    def kb(x_hbm, tab_hbm, out_hbm, idx_v, rows_v, wbuf, gsem, wsem):
        wid = lax.axis_index("s") * info.num_cores + lax.axis_index("c")
        base = wid * b_per_w
        pltpu.sync_copy(x_hbm.at[pl.ds(base, b_per_w)], idx_v)

        def gather(j, b):
            return pltpu.make_async_copy(
                tab_hbm.at[idx_v.at[pl.ds(j * CHUNK, CHUNK)]],
                rows_v.at[b],
                gsem.at[b],
            )

        def write(j, b):
            return pltpu.make_async_copy(
                wbuf.at[b],
                out_hbm.at[pl.ds(base + j * CHUNK, CHUNK)],
                wsem.at[b],
            )

        def repack(b):
            def body(i, carry):
                for k in range(8):
                    for g in range(D // 16):
                        wbuf[b, i * 8 + k, pl.ds(g * 16, 16)] = (
                            rows_v[b, i * 8 + k, pl.ds(g * 16, 16)]
                        )
                return carry

            lax.fori_loop(0, CHUNK // 8, body, 0)

        for b in range(NBUF):
            gather(b, b).start()

        @pl.loop(0, n_groups)
        def _(g):
            j0 = g * NBUF
            for b in range(NBUF):
                gather(j0 + b, b).wait()
                repack(b)
                write(j0 + b, b).start()
            for b in range(NBUF):
                write(j0 + b, b).wait()

                @pl.when(g + 1 < n_groups)
                def _():
                    gather(j0 + NBUF + b, b).start()

    return kb


def kernel(x, W_embed):
    batch, hist = x.shape
    V, D = W_embed.shape
    flat = x.reshape(batch * hist).astype(jnp.int32)
    nfull = V // 128
    tail_pad = jnp.pad(W_embed[nfull * 128:], ((0, 0), (0, 128 - D)))
    pairs = _build_detile(V, D)(W_embed.T, tail_pad)
    tab = pairs.reshape(V, D)
    out_pad = _build_gather(batch * hist, V, D)(flat, tab)
    return out_pad.reshape(batch, hist, DP)[:, :, :D]


# speedup vs baseline: 1.9841x; 1.0690x over previous
import functools

import jax
import jax.numpy as jnp
from jax import lax
from jax.experimental import pallas as pl
from jax.experimental.pallas import tpu as pltpu
from jax.experimental.pallas import tpu_sc as plsc

CHUNK = 128
NBUF = 4
DP = 128


@functools.cache
def _build_detile(V, D):
    info = plsc.get_sparse_core_info()
    nw = info.num_cores * info.num_subcores
    nblk = (V + 127) // 128
    n_per_w = (nblk + nw - 1) // nw
    mesh = plsc.VectorSubcoreMesh(core_axis_name="c", subcore_axis_name="s")

    @functools.partial(
        pl.kernel,
        out_type=jax.ShapeDtypeStruct((V // 2, 128), jnp.float32),
        mesh=mesh,
        scratch_types=[
            pltpu.VMEM((2, D, 128), jnp.float32),
            pltpu.VMEM((2, 64, 128), jnp.float32),
            pltpu.SemaphoreType.DMA((2,)),
            pltpu.SemaphoreType.DMA((2,)),
        ],
        compiler_params=pltpu.CompilerParams(use_tc_tiling_on_sc=True, needs_layout_passes=False),
    )
    def ka(wt_hbm, tail_hbm, out_hbm, stag, obuf, gsem, wsem):
        wid = lax.axis_index("s") * info.num_cores + lax.axis_index("c")
        iota = lax.iota(jnp.int32, 16)
        nfull = V // 128
        tail = V - nfull * 128
        npw = (nfull + nw - 1) // nw

        def uclamp(t):
            return jnp.minimum(wid * npw + t, nfull - 1)

        def copy_in(t, p):
            c0 = uclamp(t) * 128
            return pltpu.make_async_copy(
                wt_hbm.at[:, pl.ds(c0, 128)], stag.at[p], gsem.at[p]
            )

        def copy_out(t, p):
            return pltpu.make_async_copy(
                obuf.at[p], out_hbm.at[pl.ds(uclamp(t) * 64, 64)], wsem.at[p]
            )

        rots = [(iota + s) % 16 for s in range(16)]

        def transpose_block(p):
            def body(ib4, carry):
                ib = ib4 * 16
                r0 = ib4 * 8
                for jb in range(0, D, 16):
                    jvec = jb + iota
                    for s in range(16):
                        ivec = ib + rots[s]
                        val = plsc.load_gather(stag.at[p], [jvec, ivec])
                        rvec = r0 + (rots[s] >> 1)
                        cvec = (rots[s] & 1) * 64 + jvec
                        plsc.store_scatter(obuf.at[p], [rvec, cvec], val)
                return carry

            lax.fori_loop(0, 8, body, 0)

        def step(t, p, first):
            copy_in(t, p).wait()
            if not first:
                copy_out(t - 2, p).wait()
            transpose_block(p)
            copy_out(t, p).start()

        copy_in(0, 0).start()
        copy_in(1, 1).start()

        @pl.loop(0, (npw - 1) // 2)
        def _(h):
            for p in range(2):
                t = h * 2 + p

                @pl.when(h >= 1)
                def _():
                    copy_out(t - 2, p).wait()
                copy_in(t, p).wait()
                transpose_block(p)
                copy_out(t, p).start()

                @pl.when(t + 2 < npw)
                def _():
                    copy_in(t + 2, p).start()

        tlast = npw - 1
        copy_in(tlast, 0).wait()
        copy_out(tlast - 2, 0).wait()
        transpose_block(0)
        copy_out(tlast, 0).start()
        copy_out(tlast - 1, 1).wait()
        copy_out(tlast, 0).wait()

        if tail:
            @pl.when(wid == 0)
            def _():
                pltpu.sync_copy(tail_hbm, stag.at[0])
                for i in range(tail):
                    for g in range(D // 16):
                        obuf[0, i >> 1, pl.ds((i & 1) * 64 + g * 16, 16)] = (
                            stag[0, i, pl.ds(g * 16, 16)]
                        )
                pltpu.sync_copy(
                    obuf.at[0, pl.ds(0, tail // 2)],
                    out_hbm.at[pl.ds(nfull * 64, tail // 2)],
                )

    return ka


@functools.cache
def _build_gather(B, V, D):
    info = plsc.get_sparse_core_info()
    nw = info.num_cores * info.num_subcores
    assert B % (nw * CHUNK * NBUF) == 0
    b_per_w = B // nw
    n_groups = b_per_w // (CHUNK * NBUF)
    mesh = plsc.VectorSubcoreMesh(core_axis_name="c", subcore_axis_name="s")

    @functools.partial(
        pl.kernel,
        out_type=jax.ShapeDtypeStruct((B, DP), jnp.float32),
        mesh=mesh,
        scratch_types=[
            pltpu.VMEM((b_per_w,), jnp.int32),
            pltpu.VMEM((2, CHUNK, D), jnp.float32),
            pltpu.VMEM((NBUF, CHUNK, DP), jnp.float32),
            pltpu.SemaphoreType.DMA((2,)),
            pltpu.SemaphoreType.DMA((NBUF,)),
        ],
        compiler_params=pltpu.CompilerParams(use_tc_tiling_on_sc=False),
    )
    def kb(x_hbm, tab_hbm, out_hbm, idx_v, rows_v, wbuf, gsem, wsem):
        wid = lax.axis_index("s") * info.num_cores + lax.axis_index("c")
        base = wid * b_per_w
        n_chunks = b_per_w // CHUNK
        pltpu.sync_copy(x_hbm.at[pl.ds(base, b_per_w)], idx_v)

        def gather(j, p):
            return pltpu.make_async_copy(
                tab_hbm.at[idx_v.at[pl.ds(j * CHUNK, CHUNK)]],
                rows_v.at[p],
                gsem.at[p],
            )

        def write(j, q):
            return pltpu.make_async_copy(
                wbuf.at[q],
                out_hbm.at[pl.ds(base + j * CHUNK, CHUNK)],
                wsem.at[q],
            )

        def repack(p, q):
            def body(i, carry):
                for k in range(8):
                    for g in range(D // 16):
                        wbuf[q, i * 8 + k, pl.ds(g * 16, 16)] = (
                            rows_v[p, i * 8 + k, pl.ds(g * 16, 16)]
                        )
                return carry

            lax.fori_loop(0, CHUNK // 8, body, 0)

        gather(0, 0).start()
        gather(1, 1).start()

        @pl.loop(0, n_groups)
        def _(h):
            for q in range(NBUF):
                j = h * NBUF + q
                p = q % 2
                gather(j, p).wait()

                @pl.when(h >= 1)
                def _():
                    write(j - NBUF, q).wait()
                repack(p, q)
                write(j, q).start()

                @pl.when(j + 2 < n_chunks)
                def _():
                    gather(j + 2, p).start()

        for q in range(NBUF):
            write((n_groups - 1) * NBUF + q, q).wait()

    return kb


def kernel(x, W_embed):
    batch, hist = x.shape
    V, D = W_embed.shape
    flat = x.reshape(batch * hist).astype(jnp.int32)
    nfull = V // 128
    tail_pad = jnp.pad(W_embed[nfull * 128:], ((0, 0), (0, 128 - D)))
    pairs = _build_detile(V, D)(W_embed.T, tail_pad)
    tab = pairs.reshape(V, D)
    out_pad = _build_gather(batch * hist, V, D)(flat, tab)
    return out_pad.reshape(batch, hist, DP)[:, :, :D]


# speedup vs baseline: 2.1863x; 1.1019x over previous
import functools

import jax
import jax.numpy as jnp
from jax import lax
from jax.experimental import pallas as pl
from jax.experimental.pallas import tpu as pltpu
from jax.experimental.pallas import tpu_sc as plsc

CHUNK = 128
NBUF = 4
DP = 128


@functools.cache
def _build(B, V):
    info = plsc.get_sparse_core_info()
    nw = info.num_cores * info.num_subcores
    assert B % (nw * CHUNK * NBUF) == 0
    b_per_w = B // nw
    n_groups = b_per_w // (CHUNK * NBUF)
    mesh = plsc.VectorSubcoreMesh(core_axis_name="c", subcore_axis_name="s")

    @functools.partial(
        pl.kernel,
        out_type=jax.ShapeDtypeStruct((B, DP), jnp.float32),
        mesh=mesh,
        scratch_types=[
            pltpu.VMEM((b_per_w,), jnp.int32),
            pltpu.VMEM((NBUF, CHUNK, DP), jnp.float32),
            pltpu.SemaphoreType.DMA((NBUF,)),
            pltpu.SemaphoreType.DMA((NBUF,)),
        ],
    )
    def k(x_hbm, tab_hbm, out_hbm, idx_v, rows_v, gsem, wsem):
        wid = lax.axis_index("s") * info.num_cores + lax.axis_index("c")
        base = wid * b_per_w
        pltpu.sync_copy(x_hbm.at[pl.ds(base, b_per_w)], idx_v)

        def gather(j, b):
            return pltpu.make_async_copy(
                tab_hbm.at[idx_v.at[pl.ds(j * CHUNK, CHUNK)]],
                rows_v.at[b],
                gsem.at[b],
            )

        def write(j, b):
            return pltpu.make_async_copy(
                rows_v.at[b],
                out_hbm.at[pl.ds(base + j * CHUNK, CHUNK)],
                wsem.at[b],
            )

        for b in range(NBUF):
            gather(b, b).start()

        @pl.loop(0, n_groups)
        def _(g):
            j0 = g * NBUF
            for b in range(NBUF):
                gather(j0 + b, b).wait()
                write(j0 + b, b).start()
            for b in range(NBUF):
                write(j0 + b, b).wait()

                @pl.when(g + 1 < n_groups)
                def _():
                    gather(j0 + NBUF + b, b).start()

    return k


def kernel(x, W_embed):
    batch, hist = x.shape
    V, D = W_embed.shape
    flat = x.reshape(batch * hist).astype(jnp.int32)
    Wp = jnp.pad(W_embed, ((0, 0), (0, DP - D)))
    out_pad = _build(batch * hist, V)(flat, Wp)
    return out_pad.reshape(batch, hist, DP)[:, :, :D]
